# baseline (speedup 1.0000x reference)
import functools

import jax
import jax.numpy as jnp
from jax import lax
from jax.experimental import pallas as pl
from jax.experimental.pallas import tpu as pltpu
from jax.experimental.pallas import tpu_sc as plsc

BATCH = 16384
SEQ = 200
EMBED = 32
HIDDEN = 128
CLS = 10

NC, NS = 2, 16
NW = NC * NS
ROWS_PER_W = BATCH // NW
HALF = 256
SEQ_A = 128
SEQ_B = SEQ - SEQ_A
NBUF = 4
INV_SEQ = 1.0 / SEQ


VOCAB = 1000000
CTILES = VOCAB // 128
CREM = VOCAB - CTILES * 128


def _transpose_sc(tableT, tail2d):
    mesh = plsc.VectorSubcoreMesh(core_axis_name="c", subcore_axis_name="s")

    @functools.partial(
        pl.kernel,
        out_type=jax.ShapeDtypeStruct((VOCAB * EMBED // 128, 128),
                                      jnp.float32),
        mesh=mesh,
        scratch_types=[
            [pltpu.VMEM((32, 512), jnp.float32)] * 3,
            [pltpu.VMEM((128, 128), jnp.float32)] * 3,
            [pltpu.SemaphoreType.DMA] * 3,
            [pltpu.SemaphoreType.DMA] * 3,
        ],
        compiler_params=pltpu.CompilerParams(
            use_tc_tiling_on_sc=True, needs_layout_passes=False),
    )
    def k(t_hbm, tail_hbm, flat_hbm, in_v, out_v, in_sems, out_sems):
        wid = lax.axis_index("s") * NC + lax.axis_index("c")
        lanes = jax.lax.iota(jnp.int32, 16)
        rbase = lanes // 4
        cbase = (lanes % 4) * 32

        NSB = CTILES // 4

        def fire_in(sb, b):
            pltpu.async_copy(
                t_hbm.at[:, pl.ds(sb * 512, 512)], in_v[b], in_sems[b])

        def drain_in(b):
            pltpu.make_async_copy(
                t_hbm.at[:, pl.ds(0, 512)], in_v[b], in_sems[b]).wait()

        def drain_out(b):
            pltpu.make_async_copy(
                out_v[b], flat_hbm.at[pl.ds(0, 128)],
                out_sems[b]).wait()

        def fold(b):
            rows = [rbase + 4 * u for u in range(32)]

            @pl.loop(0, 32)
            def _f(f):
                cols = cbase + f
                for ub in range(0, 32, 8):
                    vs = [in_v[b][f, pl.ds(16 * (ub + k), 16)]
                          for k in range(8)]
                    for k in range(8):
                        plsc.store_scatter(
                            out_v[b], [rows[ub + k], cols], vs[k])

        RING = 3
        for b in range(RING):
            @pl.when(wid + 32 * b < NSB)
            def _prime():
                fire_in(wid + 32 * b, b)

        nj = NSB // NW + 2 * RING
        @pl.loop(0, nj, step=RING)
        def _blocks(jj):
            for b in range(RING):
                j = jj + b
                sb = wid + 32 * j

                @pl.when(sb < NSB)
                def _one():
                    drain_in(b)

                    @pl.when(j >= RING)
                    def _w():
                        drain_out(b)

                    fold(b)

                    @pl.when(sb + RING * NW < NSB)
                    def _next():
                        fire_in(sb + RING * NW, b)

                    pltpu.async_copy(
                        out_v[b],
                        flat_hbm.at[pl.ds(sb * 128, 128)], out_sems[b])

        for b in range(RING):
            drain_out(b)

        @pl.when(wid == NW - 1)
        def _tail():
            pltpu.sync_copy(tail_hbm, out_v[0].at[pl.ds(0, 16)])
            pltpu.sync_copy(out_v[0].at[pl.ds(0, 16)],
                            flat_hbm.at[pl.ds(CTILES * 32, 16)])

    return k(tableT, tail2d)


def _transpose_tc(tableT, perm):
    def body(in_ref, p_ref, out_ref):
        x = in_ref[...]
        ct = jnp.dot(x, p_ref[...], preferred_element_type=jnp.float32)
        out_ref[...] = jnp.concatenate(
            [jnp.transpose(ct[:, 128 * k:128 * (k + 1)]) for k in range(4)],
            axis=1)

    return pl.pallas_call(
        body,
        grid=(1954,),
        in_specs=[pl.BlockSpec((EMBED, 512), lambda i: (0, i)),
                  pl.BlockSpec((512, 512), lambda i: (0, 0))],
        out_specs=pl.BlockSpec((128, 128), lambda i: (i, 0)),
        out_shape=jax.ShapeDtypeStruct((VOCAB * EMBED // 128, 128),
                                       jnp.float32),
    )(tableT, perm)


def _pool_sc(x, table):
    mesh = plsc.VectorSubcoreMesh(core_axis_name="c", subcore_axis_name="s")

    @functools.partial(
        pl.kernel,
        out_type=jax.ShapeDtypeStruct((BATCH, EMBED), jnp.float32),
        mesh=mesh,
        scratch_types=[
            pltpu.VMEM((HALF, SEQ), jnp.int32),
            pltpu.VMEM((NBUF, SEQ, EMBED), jnp.float32),
            pltpu.VMEM((HALF, EMBED), jnp.float32),
            [pltpu.SemaphoreType.DMA] * NBUF,
        ],
        compiler_params=pltpu.CompilerParams(use_tc_tiling_on_sc=False),
    )
    def k(x_hbm, tab_hbm, out_hbm, idx_v, g_v, o_v, sems):
        wid = lax.axis_index("s") * NC + lax.axis_index("c")
        base = wid * ROWS_PER_W

        def issue(row, b):
            pltpu.async_copy(
                tab_hbm.at[idx_v.at[row, pl.ds(0, SEQ_A)]],
                g_v.at[b, pl.ds(0, SEQ_A)], sems[b])
            pltpu.async_copy(
                tab_hbm.at[idx_v.at[row, pl.ds(SEQ_A, SEQ_B)]],
                g_v.at[b, pl.ds(SEQ_A, SEQ_B)], sems[b])

        def drain(b):
            pltpu.make_async_copy(
                tab_hbm.at[idx_v.at[0, pl.ds(0, SEQ_A)]],
                g_v.at[b, pl.ds(0, SEQ_A)], sems[b]).wait()
            pltpu.make_async_copy(
                tab_hbm.at[idx_v.at[0, pl.ds(SEQ_A, SEQ_B)]],
                g_v.at[b, pl.ds(SEQ_A, SEQ_B)], sems[b]).wait()

        for half in range(ROWS_PER_W // HALF):
            hbase = base + half * HALF
            pltpu.sync_copy(x_hbm.at[pl.ds(hbase, HALF)], idx_v)
            for b in range(NBUF):
                issue(b, b)

            @pl.loop(0, HALF, step=NBUF)
            def _rows(rc):
                for b in range(NBUF):
                    r = rc + b
                    drain(b)

                    def body(i, carry):
                        a0, a1 = carry
                        return (a0 + g_v[b, i, pl.ds(0, 16)],
                                a1 + g_v[b, i, pl.ds(16, 16)])

                    a0, a1 = lax.fori_loop(
                        0, SEQ, body,
                        (jnp.zeros((16,), jnp.float32),
                         jnp.zeros((16,), jnp.float32)),
                        unroll=8)
                    o_v[r, pl.ds(0, 16)] = a0 * INV_SEQ
                    o_v[r, pl.ds(16, 16)] = a1 * INV_SEQ

                    @pl.when(rc + NBUF < HALF)
                    def _prefetch():
                        issue(r + NBUF, b)

            pltpu.sync_copy(o_v, out_hbm.at[pl.ds(hbase, HALF)])

    return k(x, table)


def _mlp_tc(pooled, W1, b1, W2, b2):
    BB = 2048

    def body(p_ref, w1_ref, b1_ref, w2_ref, b2_ref, o_ref):
        h = jnp.dot(p_ref[...], w1_ref[...],
                    preferred_element_type=jnp.float32)
        h = jnp.maximum(h + b1_ref[...], 0.0)
        o_ref[...] = jnp.dot(h, w2_ref[...],
                             preferred_element_type=jnp.float32) + b2_ref[...]

    return pl.pallas_call(
        body,
        grid=(BATCH // BB,),
        in_specs=[
            pl.BlockSpec((BB, EMBED), lambda i: (i, 0)),
            pl.BlockSpec((EMBED, HIDDEN), lambda i: (0, 0)),
            pl.BlockSpec((1, HIDDEN), lambda i: (0, 0)),
            pl.BlockSpec((HIDDEN, CLS), lambda i: (0, 0)),
            pl.BlockSpec((1, CLS), lambda i: (0, 0)),
        ],
        out_specs=pl.BlockSpec((BB, CLS), lambda i: (i, 0)),
        out_shape=jax.ShapeDtypeStruct((BATCH, CLS), jnp.float32),
    )(pooled, W1, b1.reshape(1, HIDDEN), W2, b2.reshape(1, CLS))


def kernel(x, table, W1, b1, W2, b2):
    c = jnp.arange(512)
    perm = jax.nn.one_hot(128 * (c % 4) + c // 4, 512, dtype=jnp.float32)
    table_rm = _transpose_tc(table.T, perm).reshape(VOCAB, EMBED)
    pooled = _pool_sc(x, table_rm)
    return _mlp_tc(pooled, W1, b1, W2, b2)


# speedup vs baseline: 2.1529x; 2.1529x over previous
import functools

import jax
import jax.numpy as jnp
from jax import lax
from jax.experimental import pallas as pl
from jax.experimental.pallas import tpu as pltpu
from jax.experimental.pallas import tpu_sc as plsc

BATCH = 16384
SEQ = 200
EMBED = 32
HIDDEN = 128
CLS = 10

NC, NS = 2, 16
NW = NC * NS
ROWS_PER_W = BATCH // NW
HALF = 256
SEQ_A = 128
SEQ_B = SEQ - SEQ_A
NBUF = 4
INV_SEQ = 1.0 / SEQ


def _pool_sc(x, table):
    mesh = plsc.VectorSubcoreMesh(core_axis_name="c", subcore_axis_name="s")

    @functools.partial(
        pl.kernel,
        out_type=jax.ShapeDtypeStruct((BATCH, EMBED), jnp.float32),
        mesh=mesh,
        scratch_types=[
            pltpu.VMEM((HALF, SEQ), jnp.int32),
            pltpu.VMEM((NBUF, SEQ, EMBED), jnp.float32),
            pltpu.VMEM((HALF, EMBED), jnp.float32),
            [pltpu.SemaphoreType.DMA] * NBUF,
        ],
        compiler_params=pltpu.CompilerParams(use_tc_tiling_on_sc=False),
    )
    def k(x_hbm, tab_hbm, out_hbm, idx_v, g_v, o_v, sems):
        wid = lax.axis_index("s") * NC + lax.axis_index("c")
        base = wid * ROWS_PER_W

        def issue(row, b):
            pltpu.async_copy(
                tab_hbm.at[idx_v.at[row, pl.ds(0, SEQ_A)]],
                g_v.at[b, pl.ds(0, SEQ_A)], sems[b])
            pltpu.async_copy(
                tab_hbm.at[idx_v.at[row, pl.ds(SEQ_A, SEQ_B)]],
                g_v.at[b, pl.ds(SEQ_A, SEQ_B)], sems[b])

        def drain(b):
            pltpu.make_async_copy(
                tab_hbm.at[idx_v.at[0, pl.ds(0, SEQ_A)]],
                g_v.at[b, pl.ds(0, SEQ_A)], sems[b]).wait()
            pltpu.make_async_copy(
                tab_hbm.at[idx_v.at[0, pl.ds(SEQ_A, SEQ_B)]],
                g_v.at[b, pl.ds(SEQ_A, SEQ_B)], sems[b]).wait()

        for half in range(ROWS_PER_W // HALF):
            hbase = base + half * HALF
            pltpu.sync_copy(x_hbm.at[pl.ds(hbase, HALF)], idx_v)
            for b in range(NBUF):
                issue(b, b)

            @pl.loop(0, HALF, step=NBUF)
            def _rows(rc):
                for b in range(NBUF):
                    r = rc + b
                    drain(b)

                    def body(i, carry):
                        a0, a1 = carry
                        return (a0 + g_v[b, i, pl.ds(0, 16)],
                                a1 + g_v[b, i, pl.ds(16, 16)])

                    a0, a1 = lax.fori_loop(
                        0, SEQ, body,
                        (jnp.zeros((16,), jnp.float32),
                         jnp.zeros((16,), jnp.float32)),
                        unroll=8)
                    o_v[r, pl.ds(0, 16)] = a0 * INV_SEQ
                    o_v[r, pl.ds(16, 16)] = a1 * INV_SEQ

                    @pl.when(rc + NBUF < HALF)
                    def _prefetch():
                        issue(r + NBUF, b)

            pltpu.sync_copy(o_v, out_hbm.at[pl.ds(hbase, HALF)])

    return k(x, table)


def _mlp_tc(pooled, W1, b1, W2, b2):
    BB = 2048

    def body(p_ref, w1_ref, b1_ref, w2_ref, b2_ref, o_ref):
        h = jnp.dot(p_ref[...], w1_ref[...],
                    preferred_element_type=jnp.float32)
        h = jnp.maximum(h + b1_ref[...], 0.0)
        o_ref[...] = jnp.dot(h, w2_ref[...],
                             preferred_element_type=jnp.float32) + b2_ref[...]

    return pl.pallas_call(
        body,
        grid=(BATCH // BB,),
        in_specs=[
            pl.BlockSpec((BB, EMBED), lambda i: (i, 0)),
            pl.BlockSpec((EMBED, HIDDEN), lambda i: (0, 0)),
            pl.BlockSpec((1, HIDDEN), lambda i: (0, 0)),
            pl.BlockSpec((HIDDEN, CLS), lambda i: (0, 0)),
            pl.BlockSpec((1, CLS), lambda i: (0, 0)),
        ],
        out_specs=pl.BlockSpec((BB, CLS), lambda i: (i, 0)),
        out_shape=jax.ShapeDtypeStruct((BATCH, CLS), jnp.float32),
    )(pooled, W1, b1.reshape(1, HIDDEN), W2, b2.reshape(1, CLS))


def kernel(x, table, W1, b1, W2, b2):
    pooled = _pool_sc(x, table)
    return _mlp_tc(pooled, W1, b1, W2, b2)
